# Initial kernel scaffold; baseline (speedup 1.0000x reference)
import jax
import jax.numpy as jnp
from jax.experimental import pallas as pl


def kernel(input, emb, W_ih, W_hh, b_lstm, W_if, b_if, W_out, b_out, W_fc, b_fc):
    raise NotImplementedError("write your pallas kernel here")


# speedup vs baseline: 1.2762x; 1.2762x over previous
import functools

import jax
import jax.numpy as jnp
from jax import lax
from jax.experimental import pallas as pl
from jax.experimental.pallas import tpu as pltpu
from jax.experimental.pallas import tpu_sc as plsc

MODEL = 128
NHEAD = 4
NCELLS = 512
VOCAB = 1000
MEM = 64
B, T = 8, 128
IFACE_PAD = 512


def _make_sc_gather():
    info = plsc.get_sparse_core_info()
    nc, ns = info.num_cores, info.num_subcores
    nw = nc * ns
    n_idx = T * B
    per_w = n_idx // nw
    mesh = plsc.VectorSubcoreMesh(core_axis_name="c", subcore_axis_name="s")

    @functools.partial(
        pl.kernel,
        mesh=mesh,
        out_type=jax.ShapeDtypeStruct((n_idx, MODEL), jnp.float32),
        scratch_types=[
            pltpu.VMEM((per_w,), jnp.int32),
            pltpu.VMEM((per_w, MODEL), jnp.float32),
            pltpu.SemaphoreType.DMA,
        ],
    )
    def gather(table_hbm, idx_hbm, out_hbm, idx_v, rows_v, sem):
        wid = lax.axis_index("s") * nc + lax.axis_index("c")
        base = wid * per_w
        pltpu.sync_copy(idx_hbm.at[pl.ds(base, per_w)], idx_v)
        pltpu.async_copy(table_hbm.at[idx_v], rows_v, sem).wait()
        pltpu.sync_copy(rows_v, out_hbm.at[pl.ds(base, per_w)])

    return gather


def _dot(a, b, ca, cb):
    return lax.dot_general(
        a, b, (((ca,), (cb,)), ((), ())), preferred_element_type=jnp.float32
    )


def _softplus(x):
    return jnp.maximum(x, 0.0) + jnp.log(1.0 + jnp.exp(-jnp.abs(x)))


def _dnc_body(
    xs_ref, wihx_ref, wihr_ref, whh_ref, bl_ref, wif_ref, bif_ref,
    wout_ref, bout_ref, wfc_ref, bfc_ref, out_ref,
    mT, hT, cT, rT, nrm, outs,
):
    mT[...] = jnp.zeros_like(mT)
    hT[...] = jnp.zeros_like(hT)
    cT[...] = jnp.zeros_like(cT)
    rT[...] = jnp.zeros_like(rT)
    nrm[...] = jnp.zeros_like(nrm)

    wihx = wihx_ref[...]
    wihr = wihr_ref[...]
    whh = whh_ref[...]
    bl = bl_ref[...]
    wif = wif_ref[...]
    bif = bif_ref[...]

    def step(t, carry):
        x_t = xs_ref[t]
        gates = (
            _dot(wihx, x_t, 1, 1)
            + jnp.dot(wihr, rT[...], preferred_element_type=jnp.float32)
            + jnp.dot(whh, hT[...], preferred_element_type=jnp.float32)
            + bl
        )
        ig = jax.nn.sigmoid(gates[0:MODEL])
        fg = jax.nn.sigmoid(gates[MODEL : 2 * MODEL])
        gg = jnp.tanh(gates[2 * MODEL : 3 * MODEL])
        og = jax.nn.sigmoid(gates[3 * MODEL : 4 * MODEL])
        c_new = fg * cT[...] + ig * gg
        h_new = og * jnp.tanh(c_new)
        cT[...] = c_new
        hT[...] = h_new

        iface = jnp.dot(wif, h_new, preferred_element_type=jnp.float32) + bif
        wkT = iface[NHEAD * MEM : NHEAD * MEM + MEM]
        wvT = iface[NHEAD * MEM + MEM : NHEAD * MEM + 2 * MEM]
        evT = jax.nn.sigmoid(iface[NHEAD * MEM + 2 * MEM : NHEAD * MEM + 3 * MEM])
        beta_rows = iface[NHEAD * MEM + 3 * MEM : NHEAD * MEM + 3 * MEM + 8]
        betas = _softplus(jnp.transpose(beta_rows)) + 1.0
        rbeta = betas[:, 0:NHEAD]
        wbeta = betas[:, NHEAD : NHEAD + 1]

        knrm_rows = jnp.concatenate(
            [
                jnp.sum(iface[j * MEM : (j + 1) * MEM] ** 2, axis=0, keepdims=True)
                for j in range(NHEAD + 1)
            ]
            + [jnp.zeros((3, B), jnp.float32)],
            axis=0,
        )
        knorm = jnp.sqrt(jnp.transpose(knrm_rows))
        rknorm = knorm[:, 0:NHEAD]
        wknorm = knorm[:, NHEAD : NHEAD + 1]

        m = mT[...]
        simw = jnp.sum(m * wkT[:, :, None], axis=0)
        simw = simw / ((nrm[...] + 1e-6) * (wknorm + 1e-6)) * wbeta
        mx = jnp.max(simw, axis=-1, keepdims=True)
        e = jnp.exp(simw - mx)
        ww = e / jnp.sum(e, axis=-1, keepdims=True)

        m = m * (1.0 - ww[None, :, :] * evT[:, :, None]) + ww[None, :, :] * wvT[:, :, None]
        mT[...] = m
        nrm_new = jnp.sqrt(jnp.sum(m * m, axis=0))
        nrm[...] = nrm_new

        reads = []
        for h in range(NHEAD):
            rkT = iface[h * MEM : (h + 1) * MEM]
            simr = jnp.sum(m * rkT[:, :, None], axis=0)
            simr = (
                simr
                / ((nrm_new + 1e-6) * (rknorm[:, h : h + 1] + 1e-6))
                * rbeta[:, h : h + 1]
            )
            mxr = jnp.max(simr, axis=-1, keepdims=True)
            er = jnp.exp(simr - mxr)
            wr = er / jnp.sum(er, axis=-1, keepdims=True)
            reads.append(jnp.sum(m * wr[None, :, :], axis=-1))
        r_new = jnp.concatenate(reads, axis=0)
        rT[...] = r_new

        catT = jnp.concatenate([h_new, r_new], axis=0)
        out_t = _dot(catT, wout_ref[...], 0, 1) + bout_ref[...]
        outs[t] = out_t
        return carry

    lax.fori_loop(0, T, step, 0)

    wfc = wfc_ref[...]
    bfc = bfc_ref[...]
    for b in range(B):
        src_b = outs[:, b, :]
        out_ref[b] = _dot(wfc, src_b, 1, 1) + bfc


def _recurrence(xs, wihx, wihr, whh, bl, wifp, bifp, wout, bout, wfc, bfc):
    return pl.pallas_call(
        _dnc_body,
        out_shape=jax.ShapeDtypeStruct((B, VOCAB, T), jnp.float32),
        scratch_shapes=[
            pltpu.VMEM((MEM, B, NCELLS), jnp.float32),
            pltpu.VMEM((MODEL, B), jnp.float32),
            pltpu.VMEM((MODEL, B), jnp.float32),
            pltpu.VMEM((NHEAD * MEM, B), jnp.float32),
            pltpu.VMEM((B, NCELLS), jnp.float32),
            pltpu.VMEM((T, B, MODEL), jnp.float32),
        ],
    )(xs, wihx, wihr, whh, bl, wifp, bifp, wout, bout, wfc, bfc)


def kernel(input, emb, W_ih, W_hh, b_lstm, W_if, b_if, W_out, b_out, W_fc, b_fc):
    idx = jnp.swapaxes(input, 0, 1).reshape(T * B).astype(jnp.int32)
    rows = _make_sc_gather()(emb, idx)
    xs = rows.reshape(T, B, MODEL)

    wihx = W_ih[:, :MODEL]
    wihr = W_ih[:, MODEL:]
    bl = b_lstm.reshape(-1, 1)
    iface_dim = W_if.shape[0]
    wifp = jnp.zeros((IFACE_PAD, MODEL), jnp.float32).at[:iface_dim].set(W_if)
    bifp = jnp.zeros((IFACE_PAD, 1), jnp.float32).at[:iface_dim, 0].set(b_if)
    bout = b_out.reshape(1, -1)
    bfc = b_fc.reshape(-1, 1)

    return _recurrence(xs, wihx, wihr, W_hh, bl, wifp, bifp, W_out, bout, W_fc, bfc)
